# baseline (speedup 1.0000x reference)
import jax
import jax.numpy as jnp
from jax.experimental import pallas as pl
from jax.experimental.pallas import tpu as pltpu

_D_MODEL = 4096
_VOCAB = 100000
_BATCH = 128
_BNW = 4096
_KB = 1024
_NK = _D_MODEL // _KB


def _sample_kernel(obs_ref, w_ref, b_ref, g_ref, idx_out_ref,
                   acc_ref, bestv_ref, besti_ref):
    v = pl.program_id(0)
    k = pl.program_id(1)
    nv = pl.num_programs(0)

    partial = jnp.dot(obs_ref[:], w_ref[:], preferred_element_type=jnp.float32)

    @pl.when(k == 0)
    def _():
        acc_ref[:] = partial

    @pl.when(k > 0)
    def _():
        acc_ref[:] = acc_ref[:] + partial

    @pl.when(k == _NK - 1)
    def _():
        score = g_ref[:] + (acc_ref[:] + b_ref[:])
        col = jax.lax.broadcasted_iota(jnp.int32, (_BATCH, _BNW), 1) + v * _BNW
        score = jnp.where(col < _VOCAB, score, -jnp.inf)

        local_max = jnp.max(score, axis=1, keepdims=True)
        local_arg = jnp.min(jnp.where(score == local_max, col, _VOCAB),
                            axis=1, keepdims=True).astype(jnp.int32)

        @pl.when(v == 0)
        def _():
            bestv_ref[:] = local_max
            besti_ref[:] = local_arg

        @pl.when(v > 0)
        def _():
            better = local_max > bestv_ref[:]
            bestv_ref[:] = jnp.where(better, local_max, bestv_ref[:])
            besti_ref[:] = jnp.where(better, local_arg, besti_ref[:])

        @pl.when(v == nv - 1)
        def _():
            idx_out_ref[:] = besti_ref[:]


def kernel(obs, W, b):
    g = jax.random.gumbel(jax.random.key(42), (_BATCH, _VOCAB), jnp.float32)
    grid = (pl.cdiv(_VOCAB, _BNW), _NK)
    idx = pl.pallas_call(
        _sample_kernel,
        grid=grid,
        in_specs=[
            pl.BlockSpec((_BATCH, _KB), lambda v, k: (0, k)),
            pl.BlockSpec((_KB, _BNW), lambda v, k: (k, v)),
            pl.BlockSpec((1, _BNW), lambda v, k: (0, v)),
            pl.BlockSpec((_BATCH, _BNW), lambda v, k: (0, v)),
        ],
        out_specs=pl.BlockSpec((_BATCH, 1), lambda v, k: (0, 0)),
        out_shape=jax.ShapeDtypeStruct((_BATCH, 1), jnp.int32),
        scratch_shapes=[
            pltpu.VMEM((_BATCH, _BNW), jnp.float32),
            pltpu.VMEM((_BATCH, 1), jnp.float32),
            pltpu.VMEM((_BATCH, 1), jnp.int32),
        ],
    )(obs, W, b.reshape(1, _VOCAB), g)
    return idx.reshape(_BATCH)


# speedup vs baseline: 1.0054x; 1.0054x over previous
import jax
import jax.numpy as jnp
from jax.experimental import pallas as pl
from jax.experimental.pallas import tpu as pltpu

_D_MODEL = 4096
_VOCAB = 100000
_BATCH = 128
_BNW = 4096
_KB = 1024
_NK = _D_MODEL // _KB
_NV = (_VOCAB + _BNW - 1) // _BNW


def _sample_kernel(obs_ref, w_ref, b_ref, g_ref, maxv_ref, argv_ref, acc_ref):
    v = pl.program_id(0)
    k = pl.program_id(1)

    partial = jnp.dot(obs_ref[:, pl.ds(k * _KB, _KB)], w_ref[:],
                      preferred_element_type=jnp.float32)

    @pl.when(k == 0)
    def _():
        acc_ref[:] = partial

    @pl.when(k > 0)
    def _():
        acc_ref[:] = acc_ref[:] + partial

    @pl.when(k == _NK - 1)
    def _():
        score = g_ref[:] + (acc_ref[:] + b_ref[:])
        col = jax.lax.broadcasted_iota(jnp.int32, (_BATCH, _BNW), 1) + v * _BNW
        score = jnp.where(col < _VOCAB, score, -jnp.inf)

        local_max = jnp.max(score, axis=1, keepdims=True)
        local_arg = jnp.min(jnp.where(score == local_max, col, _VOCAB),
                            axis=1, keepdims=True).astype(jnp.int32)
        maxv_ref[:] = jnp.broadcast_to(local_max, (_BATCH, 128))
        argv_ref[:] = jnp.broadcast_to(local_arg, (_BATCH, 128))


def kernel(obs, W, b):
    g = jax.random.gumbel(jax.random.key(42), (_BATCH, _VOCAB), jnp.float32)
    grid = (_NV, _NK)
    maxv, argv = pl.pallas_call(
        _sample_kernel,
        grid=grid,
        in_specs=[
            pl.BlockSpec((_BATCH, _D_MODEL), lambda v, k: (0, 0)),
            pl.BlockSpec((_KB, _BNW), lambda v, k: (k, v)),
            pl.BlockSpec((1, _BNW), lambda v, k: (0, v)),
            pl.BlockSpec((_BATCH, _BNW), lambda v, k: (0, v)),
        ],
        out_specs=[
            pl.BlockSpec((_BATCH, 128), lambda v, k: (0, v)),
            pl.BlockSpec((_BATCH, 128), lambda v, k: (0, v)),
        ],
        out_shape=[
            jax.ShapeDtypeStruct((_BATCH, _NV * 128), jnp.float32),
            jax.ShapeDtypeStruct((_BATCH, _NV * 128), jnp.int32),
        ],
        scratch_shapes=[
            pltpu.VMEM((_BATCH, _BNW), jnp.float32),
        ],
        compiler_params=pltpu.CompilerParams(
            dimension_semantics=("parallel", "arbitrary"),
        ),
    )(obs, W, b.reshape(1, _VOCAB), g)
    maxv = maxv[:, ::128]
    argv = argv[:, ::128]
    best = jnp.max(maxv, axis=1, keepdims=True)
    samples = jnp.min(jnp.where(maxv == best, argv, _VOCAB), axis=1)
    return samples.astype(jnp.int32)
